# baseline (speedup 1.0000x reference)
import jax
import jax.numpy as jnp
from jax.experimental import pallas as pl
from jax.experimental.pallas import tpu as pltpu

_NBUF = 4


def _make_kernel(bm, nblk, kmain, n):
    ktail = n - kmain

    def _geo_conv_kernel(x_ref, adj_ref, w_ref, o_ref, buf_a, buf_b, sem_a, sem_b):
        i = pl.program_id(0)

        def copies(b):
            slot = jax.lax.rem(b, _NBUF)
            ca = pltpu.make_async_copy(
                adj_ref.at[pl.ds(b * bm, bm), pl.ds(0, kmain)],
                buf_a.at[slot], sem_a.at[slot])
            cb = pltpu.make_async_copy(
                adj_ref.at[pl.ds(b * bm, bm), pl.ds(kmain, ktail)],
                buf_b.at[slot], sem_b.at[slot])
            return ca, cb

        def start(b):
            ca, cb = copies(b)
            ca.start()
            cb.start()

        @pl.when(i == 0)
        def _():
            for k in range(min(_NBUF, nblk)):
                start(k)

        @pl.when(jnp.logical_and(i > 0, i + _NBUF - 1 < nblk))
        def _():
            start(i + _NBUF - 1)

        slot = jax.lax.rem(i, _NBUF)
        ca, cb = copies(i)
        ca.wait()
        cb.wait()
        y = jax.lax.dot_general(
            buf_a[slot], x_ref[pl.ds(0, kmain), :],
            dimension_numbers=(((1,), (0,)), ((), ())),
            preferred_element_type=jnp.float32,
        ) + jax.lax.dot_general(
            buf_b[slot], x_ref[pl.ds(kmain, ktail), :],
            dimension_numbers=(((1,), (0,)), ((), ())),
            preferred_element_type=jnp.float32,
        )
        o_ref[...] = 0.75 * y + jax.lax.dot_general(
            y, w_ref[...],
            dimension_numbers=(((1,), (1,)), ((), ())),
            preferred_element_type=jnp.float32,
        )

    return _geo_conv_kernel


def kernel(x, adj, weight):
    n, d = x.shape
    bm = 200 if n % 200 == 0 else n
    nblk = n // bm
    kmain = (n // 128) * 128 if n > 128 else n
    return pl.pallas_call(
        _make_kernel(bm, nblk, kmain, n),
        grid=(nblk,),
        in_specs=[
            pl.BlockSpec((n, d), lambda i: (0, 0)),
            pl.BlockSpec(memory_space=pl.ANY),
            pl.BlockSpec((d, d), lambda i: (0, 0)),
        ],
        out_specs=pl.BlockSpec((bm, d), lambda i: (i, 0)),
        out_shape=jax.ShapeDtypeStruct((n, d), jnp.float32),
        scratch_shapes=[
            pltpu.VMEM((_NBUF, bm, kmain), jnp.float32),
            pltpu.VMEM((_NBUF, bm, n - kmain if n > kmain else 1), jnp.float32),
            pltpu.SemaphoreType.DMA((_NBUF,)),
            pltpu.SemaphoreType.DMA((_NBUF,)),
        ],
        compiler_params=pltpu.CompilerParams(
            dimension_semantics=("arbitrary",),
        ),
    )(x, adj, weight)


# speedup vs baseline: 1.0127x; 1.0127x over previous
import jax
import jax.numpy as jnp
from jax.experimental import pallas as pl
from jax.experimental.pallas import tpu as pltpu


def _geo_conv_kernel(x_ref, adj_ref, w_ref, o_ref):
    y = jax.lax.dot_general(
        adj_ref[...], x_ref[...],
        dimension_numbers=(((1,), (0,)), ((), ())),
        preferred_element_type=jnp.float32,
    )
    o_ref[...] = 0.75 * y + jax.lax.dot_general(
        y, w_ref[...],
        dimension_numbers=(((1,), (1,)), ((), ())),
        preferred_element_type=jnp.float32,
    )


def kernel(x, adj, weight):
    n, d = x.shape
    bm = 400 if n % 400 == 0 else n
    grid = (n // bm,)
    return pl.pallas_call(
        _geo_conv_kernel,
        grid=grid,
        in_specs=[
            pl.BlockSpec((n, d), lambda i: (0, 0)),
            pl.BlockSpec((bm, n), lambda i: (i, 0)),
            pl.BlockSpec((d, d), lambda i: (0, 0)),
        ],
        out_specs=pl.BlockSpec((bm, d), lambda i: (i, 0)),
        out_shape=jax.ShapeDtypeStruct((n, d), jnp.float32),
        compiler_params=pltpu.CompilerParams(
            dimension_semantics=("parallel",),
        ),
    )(x, adj, weight)


# speedup vs baseline: 1.0128x; 1.0002x over previous
import jax
import jax.numpy as jnp
from jax.experimental import pallas as pl
from jax.experimental.pallas import tpu as pltpu


def _geo_conv_kernel(adj_ref, x_ref, w_ref, o_ref):
    y = jax.lax.dot_general(
        adj_ref[...], x_ref[...],
        dimension_numbers=(((1,), (0,)), ((), ())),
        preferred_element_type=jnp.float32,
    )
    o_ref[...] = 0.75 * y + jax.lax.dot_general(
        y, w_ref[...],
        dimension_numbers=(((1,), (1,)), ((), ())),
        preferred_element_type=jnp.float32,
    )


def kernel(x, adj, weight):
    n, d = x.shape
    bm = 400 if n % 400 == 0 else n
    grid = (n // bm,)
    return pl.pallas_call(
        _geo_conv_kernel,
        grid=grid,
        in_specs=[
            pl.BlockSpec((bm, n), lambda i: (i, 0)),
            pl.BlockSpec((n, d), lambda i: (0, 0)),
            pl.BlockSpec((d, d), lambda i: (0, 0)),
        ],
        out_specs=pl.BlockSpec((bm, d), lambda i: (i, 0)),
        out_shape=jax.ShapeDtypeStruct((n, d), jnp.float32),
        compiler_params=pltpu.CompilerParams(
            dimension_semantics=("parallel",),
        ),
    )(adj, x, weight)
